# baseline (speedup 1.0000x reference)
import functools

import jax
import jax.numpy as jnp
from jax import lax
from jax.experimental import pallas as pl
from jax.experimental.pallas import tpu as pltpu
from jax.experimental.pallas import tpu_sc as plsc

N = 10000
E = 320000
D = 128
G = 64
NP = 10240
CB = 80
CE = 80
NC = 2
NS = 16
NW = NC * NS
EW = E // NW
ED = E // NS
NCH = EW // CE
ROWS_PER_TILE = NP // NS
ROWS_PER_WORKER = NP // NW

_F32 = jnp.float32
_I32 = jnp.int32


def _mesh():
    return plsc.VectorSubcoreMesh(core_axis_name="c", subcore_axis_name="s")


def _rsqrt_newton(d):
    i = lax.bitcast_convert_type(d, _I32)
    i = jnp.int32(0x5F3759DF) - (i >> 1)
    y = lax.bitcast_convert_type(i, _F32)
    for _ in range(3):
        y = y * (1.5 - 0.5 * d * y * y)
    return y


def _deg_body(dst1, dinv_out, didx_v, ones_v, zv, degbuf_v, dv, deg_sh, sdeg):
    c = lax.axis_index("c")
    s = lax.axis_index("s")
    wid = s * NC + c

    @pl.loop(0, 40)
    def _zero_zv(k):
        zv[pl.ds(k * 16, 16)] = jnp.zeros((16,), _F32)

    @pl.loop(0, 5)
    def _ones(k):
        ones_v[pl.ds(k * 16, 16)] = jnp.ones((16,), _F32)

    pltpu.sync_copy(zv, deg_sh.at[pl.ds(s * ROWS_PER_TILE, ROWS_PER_TILE)])
    pltpu.sync_copy(dst1.at[pl.ds(s * ED, ED)], didx_v)
    plsc.subcore_barrier()

    @pl.loop(0, ED // CB // 10)
    def _scatter(t):
        for p in range(10):
            j = t * 10 + p
            pltpu.async_copy(ones_v, deg_sh.at[didx_v.at[pl.ds(j * CB, CB)]],
                             sdeg, add=True)
        for p in range(10):
            pltpu.make_async_copy(
                ones_v, deg_sh.at[didx_v.at[pl.ds(0, CB)]], sdeg).wait()

    plsc.subcore_barrier()

    base = wid * ROWS_PER_WORKER
    pltpu.sync_copy(deg_sh.at[pl.ds(base, ROWS_PER_WORKER)], degbuf_v)

    @pl.loop(0, ROWS_PER_WORKER // 16)
    def _conv(k):
        d = degbuf_v[pl.ds(k * 16, 16)] + 1.0
        dv[pl.ds(k * 16, 16)] = _rsqrt_newton(d)

    pltpu.sync_copy(dv, dinv_out.at[pl.ds(base, ROWS_PER_WORKER)])


def _sc_deg(dst1):
    f = pl.kernel(
        _deg_body,
        out_type=jax.ShapeDtypeStruct((NP,), _F32),
        mesh=_mesh(),
        scratch_types=[
            pltpu.VMEM((ED,), _I32),
            pltpu.VMEM((CB,), _F32),
            pltpu.VMEM((ROWS_PER_TILE,), _F32),
            pltpu.VMEM((ROWS_PER_WORKER,), _F32),
            pltpu.VMEM((ROWS_PER_WORKER,), _F32),
            pltpu.VMEM_SHARED((NP,), _F32),
            pltpu.SemaphoreType.DMA,
        ],
    )
    return f(dst1)


_SEG = ((0, 64), (64, 61))


def _edge_body(hw, src1, dst1, agg0, agg1, sidx, didx,
               buf0, buf1, buf2, agg_sh,
               sg0, sg1, sg2, ss0, ss1, ss2):
    c = lax.axis_index("c")
    s = lax.axis_index("s")
    wid = s * NC + c
    bufs = (buf0, buf1, buf2)
    sg = (sg0, sg1, sg2)
    ss = (ss0, ss1, ss2)

    @pl.loop(0, CE)
    def _zero_rows(r):
        for k in range(D // 16):
            buf0[r, pl.ds(k * 16, 16)] = jnp.zeros((16,), _F32)

    @pl.loop(0, ROWS_PER_TILE // CE)
    def _zero_agg(k):
        pltpu.sync_copy(buf0, agg_sh.at[pl.ds(s * ROWS_PER_TILE + k * CE, CE)])

    base = wid * EW
    plsc.subcore_barrier()

    def start_g(j, p):
        pltpu.async_copy(hw.at[sidx.at[pl.ds(j * CE, CE)]], bufs[p], sg[p])

    def wait_g(p):
        pltpu.make_async_copy(hw.at[sidx.at[pl.ds(0, CE)]], bufs[p],
                              sg[p]).wait()

    def start_s(j, p):
        pltpu.async_copy(bufs[p], agg_sh.at[didx.at[pl.ds(j * CE, CE)]],
                         ss[p], add=True)

    def wait_s(p):
        pltpu.make_async_copy(bufs[p], agg_sh.at[didx.at[pl.ds(0, CE)]],
                              ss[p]).wait()

    for cb0, nch in _SEG:
        ln = nch * CE
        pltpu.sync_copy(src1.at[pl.ds(base + cb0 * CE, ln)],
                        sidx.at[pl.ds(0, ln)])
        pltpu.sync_copy(dst1.at[pl.ds(base + cb0 * CE, ln)],
                        didx.at[pl.ds(0, ln)])
        start_g(0, 0)
        start_g(1, 1)
        wait_g(0); start_s(0, 0); start_g(2, 2)

        @pl.loop(0, (nch - 4) // 3)
        def _main(t):
            j0 = t * 3
            for i, (p, q) in enumerate(((1, 0), (2, 1), (0, 2))):
                j = j0 + 1 + i
                wait_g(p); start_s(j, p)
                wait_s(q); start_g(j + 2, q)

        for j in range(nch - 3, nch):
            p = j % 3
            wait_g(p); start_s(j, p)
            if j + 2 < nch:
                q = (j + 2) % 3
                wait_s(q); start_g(j + 2, q)
        for p in range(3):
            wait_s(p)

    plsc.subcore_barrier()

    src_slice = agg_sh.at[pl.ds(s * ROWS_PER_TILE, ROWS_PER_TILE)]

    @pl.when(c == 0)
    def _w0():
        pltpu.sync_copy(src_slice, agg0.at[pl.ds(s * ROWS_PER_TILE, ROWS_PER_TILE)])

    @pl.when(c == 1)
    def _w1():
        pltpu.sync_copy(src_slice, agg1.at[pl.ds(s * ROWS_PER_TILE, ROWS_PER_TILE)])


def _sc_edge(hw, src1, dst1):
    f = pl.kernel(
        _edge_body,
        out_type=(jax.ShapeDtypeStruct((NP, D), _F32),
                  jax.ShapeDtypeStruct((NP, D), _F32)),
        mesh=_mesh(),
        scratch_types=[
            pltpu.VMEM((_SEG[0][1] * CE,), _I32),
            pltpu.VMEM((_SEG[0][1] * CE,), _I32),
            pltpu.VMEM((CE, D), _F32),
            pltpu.VMEM((CE, D), _F32),
            pltpu.VMEM((CE, D), _F32),
            pltpu.VMEM_SHARED((NP, D), _F32),
        ] + [pltpu.SemaphoreType.DMA] * 6,
    )
    return f(hw, src1, dst1)


_PR = NP // D
_TB = NP


def _prep_positions(b2, bp2):
    qm = (b2 != bp2).astype(_F32)
    rr = lax.broadcasted_iota(_I32, (D, D), 0)
    cc = lax.broadcasted_iota(_I32, (D, D), 1)
    U = (rr <= cc).astype(_F32)
    r1 = lax.broadcasted_iota(_I32, (_PR, _PR), 0)
    r2 = lax.broadcasted_iota(_I32, (_PR, _PR), 1)
    S = (r2 < r1).astype(_F32)
    lane = lax.broadcasted_iota(_I32, (_PR, D), 1)

    def prefix(m):
        P = _dot(m, U)
        off = _dot(S, P[:, D - 1:D])
        return P + off

    qpos = prefix(qm) - 1.0
    qdump = (G + (lane % G)).astype(_F32)
    qt = jnp.where(qm > 0, qpos, qdump).astype(_I32)

    tm = 1.0 - qm
    tpos = prefix(tm) - 1.0
    tdump = ((N - G) + (lane % (_TB - (N - G)))).astype(_F32)
    tt = jnp.where(jnp.logical_and(tm > 0, tpos < (N - G)),
                   tpos, tdump).astype(_I32)
    return qt, tt


_SEL_PER_TILE = 640


def _select_body(logits, qt, tt, qfill, tool_out, query_out,
                 lv, qtv, ttv, tbv, qfv, qb_sh, tb_sh):
    c = lax.axis_index("c")
    s = lax.axis_index("s")

    @pl.when(c == 0)
    def _work():
        base = s * _SEL_PER_TILE
        pltpu.sync_copy(logits.at[pl.ds(base, _SEL_PER_TILE)], lv)
        pltpu.sync_copy(qt.at[pl.ds(base, _SEL_PER_TILE)], qtv)
        pltpu.sync_copy(tt.at[pl.ds(base, _SEL_PER_TILE)], ttv)

        @pl.when(s == 0)
        def _init():
            pltpu.sync_copy(qfill.at[0], qfv)
            pltpu.sync_copy(qfv, qb_sh)

        plsc.subcore_barrier()
        nch = jnp.where(s == NS - 1, 5, _SEL_PER_TILE // CB)

        @pl.loop(0, nch)
        def _scatter(j):
            sl = pl.ds(j * CB, CB)
            pltpu.sync_copy(lv.at[sl], qb_sh.at[qtv.at[sl]])
            pltpu.sync_copy(lv.at[sl], tb_sh.at[ttv.at[sl]])

        plsc.subcore_barrier()

        @pl.when(s == 0)
        def _out():
            pltpu.sync_copy(qb_sh.at[pl.ds(0, G)], qfv.at[pl.ds(0, G)])
            pltpu.sync_copy(qfv.at[pl.ds(0, G)], query_out)
            pltpu.sync_copy(tb_sh.at[pl.ds(0, N - G)], tbv)
            pltpu.sync_copy(tbv, tool_out)


def _sc_select(logits, qt, tt, qfill):
    f = pl.kernel(
        _select_body,
        out_type=(jax.ShapeDtypeStruct((N - G,), _F32),
                  jax.ShapeDtypeStruct((G,), _F32)),
        mesh=_mesh(),
        scratch_types=[
            pltpu.VMEM((_SEL_PER_TILE,), _F32),
            pltpu.VMEM((_SEL_PER_TILE,), _I32),
            pltpu.VMEM((_SEL_PER_TILE,), _I32),
            pltpu.VMEM((N - G,), _F32),
            pltpu.VMEM((D,), _F32),
            pltpu.VMEM_SHARED((2 * G,), _F32),
            pltpu.VMEM_SHARED((_TB,), _F32),
        ],
    )
    return f(logits, qt, tt, qfill)


_BM = 512


def _dot(a, b):
    return lax.dot_general(a, b, (((1,), (0,)), ((), ())),
                           preferred_element_type=_F32)


def _tc1a_body(x_r, wa_r, ba_r, w0_r, b2_r, bp2_r, h_r, hw_r, qt_r, tt_r):
    x = x_r[...]
    h = x + _dot(x, wa_r[...]) + ba_r[...]
    h_r[...] = h
    hw_r[...] = _dot(h, w0_r[...])

    @pl.when(pl.program_id(0) == 0)
    def _prep():
        qt, tt = _prep_positions(b2_r[...], bp2_r[...])
        qt_r[...] = qt
        tt_r[...] = tt


def _tc1a(xp, Wa, ba, W0, b2, bp2):
    grid = (NP // _BM,)
    return pl.pallas_call(
        _tc1a_body,
        grid=grid,
        in_specs=[
            pl.BlockSpec((_BM, D), lambda i: (i, 0)),
            pl.BlockSpec((D, D), lambda i: (0, 0)),
            pl.BlockSpec((1, D), lambda i: (0, 0)),
            pl.BlockSpec((D, D), lambda i: (0, 0)),
            pl.BlockSpec((_PR, D), lambda i: (0, 0)),
            pl.BlockSpec((_PR, D), lambda i: (0, 0)),
        ],
        out_specs=(pl.BlockSpec((_BM, D), lambda i: (i, 0)),
                   pl.BlockSpec((_BM, D), lambda i: (i, 0)),
                   pl.BlockSpec((_PR, D), lambda i: (0, 0)),
                   pl.BlockSpec((_PR, D), lambda i: (0, 0))),
        out_shape=(jax.ShapeDtypeStruct((NP, D), _F32),
                   jax.ShapeDtypeStruct((NP, D), _F32),
                   jax.ShapeDtypeStruct((_PR, D), _I32),
                   jax.ShapeDtypeStruct((_PR, D), _I32)),
    )(xp, Wa, ba, W0, b2, bp2)


def _tc1b_body(hw_r, dv_r, o_r):
    o_r[...] = hw_r[...] * dv_r[...]


def _tc1b(hwraw, dinvc):
    grid = (NP // _BM,)
    return pl.pallas_call(
        _tc1b_body,
        grid=grid,
        in_specs=[
            pl.BlockSpec((_BM, D), lambda i: (i, 0)),
            pl.BlockSpec((_BM, 1), lambda i: (i, 0)),
        ],
        out_specs=pl.BlockSpec((_BM, D), lambda i: (i, 0)),
        out_shape=jax.ShapeDtypeStruct((NP, D), _F32),
    )(hwraw, dinvc)


def _tc2_body(h_r, hwp_r, a0_r, a1_r, dv_r, b_r, w_r, h1_r, hw1_r):
    dv = dv_r[...]
    g = dv * (a0_r[...] + a1_r[...] + hwp_r[...]) + b_r[...]
    g = jnp.maximum(g, 0.0)
    h1 = h_r[...] + g
    h1_r[...] = h1
    hw1_r[...] = _dot(h1, w_r[...]) * dv


def _tc2(h, hwp, a0, a1, dinvc, b, W):
    grid = (NP // _BM,)
    return pl.pallas_call(
        _tc2_body,
        grid=grid,
        in_specs=[
            pl.BlockSpec((_BM, D), lambda i: (i, 0)),
            pl.BlockSpec((_BM, D), lambda i: (i, 0)),
            pl.BlockSpec((_BM, D), lambda i: (i, 0)),
            pl.BlockSpec((_BM, D), lambda i: (i, 0)),
            pl.BlockSpec((_BM, 1), lambda i: (i, 0)),
            pl.BlockSpec((1, D), lambda i: (0, 0)),
            pl.BlockSpec((D, D), lambda i: (0, 0)),
        ],
        out_specs=(pl.BlockSpec((_BM, D), lambda i: (i, 0)),
                   pl.BlockSpec((_BM, D), lambda i: (i, 0))),
        out_shape=(jax.ShapeDtypeStruct((NP, D), _F32),
                   jax.ShapeDtypeStruct((NP, D), _F32)),
    )(h, hwp, a0, a1, dinvc, b, W)


def _tc3_body(h1_r, hw1_r, a0_r, a1_r, dv_r, b_r, wo_r, bo_r, lg_r, qf_r):
    dv = dv_r[...]
    g = dv * (a0_r[...] + a1_r[...] + hw1_r[...]) + b_r[...]
    g = jnp.maximum(g, 0.0)
    h2 = h1_r[...] + g
    lg = _dot(h2, wo_r[...]) + bo_r[...]
    lg_r[...] = lg

    @pl.when(pl.program_id(0) == 0)
    def _qf():
        qf_r[...] = lg[0:1, 0:1] * jnp.ones((1, D), _F32)


def _tc3(h1, hwp1, a0, a1, dinvc, b, Wout, bout):
    grid = (NP // _BM,)
    return pl.pallas_call(
        _tc3_body,
        grid=grid,
        in_specs=[
            pl.BlockSpec((_BM, D), lambda i: (i, 0)),
            pl.BlockSpec((_BM, D), lambda i: (i, 0)),
            pl.BlockSpec((_BM, D), lambda i: (i, 0)),
            pl.BlockSpec((_BM, D), lambda i: (i, 0)),
            pl.BlockSpec((_BM, 1), lambda i: (i, 0)),
            pl.BlockSpec((1, D), lambda i: (0, 0)),
            pl.BlockSpec((D, 1), lambda i: (0, 0)),
            pl.BlockSpec((1, 1), lambda i: (0, 0)),
        ],
        out_specs=(pl.BlockSpec((_BM, 1), lambda i: (i, 0)),
                   pl.BlockSpec((1, D), lambda i: (0, 0))),
        out_shape=(jax.ShapeDtypeStruct((NP, 1), _F32),
                   jax.ShapeDtypeStruct((1, D), _F32)),
    )(h1, hwp1, a0, a1, dinvc, b, Wout, bout)


def kernel(x, W_align, b_align, W_g0, b_g0, W_g1, b_g1, W_out, b_out,
           edge_index, batch_idx, num_graphs):
    xp = jnp.pad(x, ((0, NP - N), (0, 0)))
    src1 = edge_index[0]
    dst1 = edge_index[1]
    bpad = jnp.pad(batch_idx, (0, NP - N), mode="edge")
    bprev = jnp.concatenate(
        [jnp.full((1,), -1, batch_idx.dtype), bpad[:-1]])

    dinv = _sc_deg(dst1)
    h, hw0raw, qt2, tt2 = _tc1a(xp, W_align, b_align.reshape(1, D), W_g0,
                                bpad.reshape(_PR, D), bprev.reshape(_PR, D))

    dinvc = dinv.reshape(NP, 1)
    hwp0 = _tc1b(hw0raw, dinvc)
    a00, a01 = _sc_edge(hwp0, src1, dst1)
    h1, hwp1 = _tc2(h, hwp0, a00, a01, dinvc, b_g0.reshape(1, D), W_g1)
    a10, a11 = _sc_edge(hwp1, src1, dst1)
    logits, qfill = _tc3(h1, hwp1, a10, a11, dinvc, b_g1.reshape(1, D),
                         W_out, b_out.reshape(1, 1))

    tool, query = _sc_select(logits.reshape(NP), qt2.reshape(NP),
                             tt2.reshape(NP), qfill)
    return (tool, query)


# speedup vs baseline: 1.0125x; 1.0125x over previous
import functools

import jax
import jax.numpy as jnp
from jax import lax
from jax.experimental import pallas as pl
from jax.experimental.pallas import tpu as pltpu
from jax.experimental.pallas import tpu_sc as plsc

N = 10000
E = 320000
D = 128
G = 64
NP = 10240
CB = 80
CE = 80
NC = 2
NS = 16
NW = NC * NS
EW = E // NW
ED = E // NS
NCH = EW // CE
ROWS_PER_TILE = NP // NS
ROWS_PER_WORKER = NP // NW

_F32 = jnp.float32
_I32 = jnp.int32


def _mesh():
    return plsc.VectorSubcoreMesh(core_axis_name="c", subcore_axis_name="s")


def _rsqrt_newton(d):
    i = lax.bitcast_convert_type(d, _I32)
    i = jnp.int32(0x5F3759DF) - (i >> 1)
    y = lax.bitcast_convert_type(i, _F32)
    for _ in range(3):
        y = y * (1.5 - 0.5 * d * y * y)
    return y


def _deg_body(dst1, dinv_out, didx_v, ones_v, zv, degbuf_v, dv, deg_sh, sdeg):
    c = lax.axis_index("c")
    s = lax.axis_index("s")
    wid = s * NC + c

    @pl.loop(0, 40)
    def _zero_zv(k):
        zv[pl.ds(k * 16, 16)] = jnp.zeros((16,), _F32)

    @pl.loop(0, 5)
    def _ones(k):
        ones_v[pl.ds(k * 16, 16)] = jnp.ones((16,), _F32)

    pltpu.sync_copy(zv, deg_sh.at[pl.ds(s * ROWS_PER_TILE, ROWS_PER_TILE)])
    pltpu.sync_copy(dst1.at[pl.ds(s * ED, ED)], didx_v)
    plsc.subcore_barrier()

    @pl.loop(0, ED // CB // 10)
    def _scatter(t):
        for p in range(10):
            j = t * 10 + p
            pltpu.async_copy(ones_v, deg_sh.at[didx_v.at[pl.ds(j * CB, CB)]],
                             sdeg, add=True)
        for p in range(10):
            pltpu.make_async_copy(
                ones_v, deg_sh.at[didx_v.at[pl.ds(0, CB)]], sdeg).wait()

    plsc.subcore_barrier()

    base = wid * ROWS_PER_WORKER
    pltpu.sync_copy(deg_sh.at[pl.ds(base, ROWS_PER_WORKER)], degbuf_v)

    @pl.loop(0, ROWS_PER_WORKER // 16)
    def _conv(k):
        d = degbuf_v[pl.ds(k * 16, 16)] + 1.0
        dv[pl.ds(k * 16, 16)] = _rsqrt_newton(d)

    pltpu.sync_copy(dv, dinv_out.at[pl.ds(base, ROWS_PER_WORKER)])


def _sc_deg(dst1):
    f = pl.kernel(
        _deg_body,
        out_type=jax.ShapeDtypeStruct((NP,), _F32),
        mesh=_mesh(),
        scratch_types=[
            pltpu.VMEM((ED,), _I32),
            pltpu.VMEM((CB,), _F32),
            pltpu.VMEM((ROWS_PER_TILE,), _F32),
            pltpu.VMEM((ROWS_PER_WORKER,), _F32),
            pltpu.VMEM((ROWS_PER_WORKER,), _F32),
            pltpu.VMEM_SHARED((NP,), _F32),
            pltpu.SemaphoreType.DMA,
        ],
    )
    return f(dst1)


_SEG = ((0, 64), (64, 61))


def _edge_body(hw, src1, dst1, agg0, agg1, sidx, didx,
               buf0, buf1, buf2, agg_sh,
               sg0, sg1, sg2, ss0, ss1, ss2):
    c = lax.axis_index("c")
    s = lax.axis_index("s")
    wid = s * NC + c
    bufs = (buf0, buf1, buf2)
    sg = (sg0, sg1, sg2)
    ss = (ss0, ss1, ss2)

    @pl.loop(0, CE)
    def _zero_rows(r):
        for k in range(D // 16):
            buf0[r, pl.ds(k * 16, 16)] = jnp.zeros((16,), _F32)

    @pl.loop(0, ROWS_PER_TILE // CE)
    def _zero_agg(k):
        pltpu.async_copy(buf0,
                         agg_sh.at[pl.ds(s * ROWS_PER_TILE + k * CE, CE)],
                         ss0)

    @pl.loop(0, ROWS_PER_TILE // CE)
    def _zero_wait(k):
        pltpu.make_async_copy(
            buf0, agg_sh.at[pl.ds(s * ROWS_PER_TILE, CE)], ss0).wait()

    base = wid * EW

    def start_g(j, p):
        pltpu.async_copy(hw.at[sidx.at[pl.ds(j * CE, CE)]], bufs[p], sg[p])

    def wait_g(p):
        pltpu.make_async_copy(hw.at[sidx.at[pl.ds(0, CE)]], bufs[p],
                              sg[p]).wait()

    def start_s(j, p):
        pltpu.async_copy(bufs[p], agg_sh.at[didx.at[pl.ds(j * CE, CE)]],
                         ss[p], add=True)

    def wait_s(p):
        pltpu.make_async_copy(bufs[p], agg_sh.at[didx.at[pl.ds(0, CE)]],
                              ss[p]).wait()

    for seg, (cb0, nch) in enumerate(_SEG):
        ln = nch * CE
        pltpu.sync_copy(src1.at[pl.ds(base + cb0 * CE, ln)],
                        sidx.at[pl.ds(0, ln)])
        pltpu.sync_copy(dst1.at[pl.ds(base + cb0 * CE, ln)],
                        didx.at[pl.ds(0, ln)])
        start_g(0, 0)
        start_g(1, 1)
        if seg == 0:
            plsc.subcore_barrier()
        wait_g(0); start_s(0, 0); start_g(2, 2)

        @pl.loop(0, (nch - 4) // 3)
        def _main(t):
            j0 = t * 3
            for i, (p, q) in enumerate(((1, 0), (2, 1), (0, 2))):
                j = j0 + 1 + i
                wait_g(p); start_s(j, p)
                wait_s(q); start_g(j + 2, q)

        for j in range(nch - 3, nch):
            p = j % 3
            wait_g(p); start_s(j, p)
            if j + 2 < nch:
                q = (j + 2) % 3
                wait_s(q); start_g(j + 2, q)
        for p in range(3):
            wait_s(p)

    plsc.subcore_barrier()

    src_slice = agg_sh.at[pl.ds(s * ROWS_PER_TILE, ROWS_PER_TILE)]

    @pl.when(c == 0)
    def _w0():
        pltpu.sync_copy(src_slice, agg0.at[pl.ds(s * ROWS_PER_TILE, ROWS_PER_TILE)])

    @pl.when(c == 1)
    def _w1():
        pltpu.sync_copy(src_slice, agg1.at[pl.ds(s * ROWS_PER_TILE, ROWS_PER_TILE)])


def _sc_edge(hw, src1, dst1):
    f = pl.kernel(
        _edge_body,
        out_type=(jax.ShapeDtypeStruct((NP, D), _F32),
                  jax.ShapeDtypeStruct((NP, D), _F32)),
        mesh=_mesh(),
        scratch_types=[
            pltpu.VMEM((_SEG[0][1] * CE,), _I32),
            pltpu.VMEM((_SEG[0][1] * CE,), _I32),
            pltpu.VMEM((CE, D), _F32),
            pltpu.VMEM((CE, D), _F32),
            pltpu.VMEM((CE, D), _F32),
            pltpu.VMEM_SHARED((NP, D), _F32),
        ] + [pltpu.SemaphoreType.DMA] * 6,
    )
    return f(hw, src1, dst1)


_PR = NP // D
_TB = NP


def _prep_positions(b2, bp2):
    qm = (b2 != bp2).astype(_F32)
    rr = lax.broadcasted_iota(_I32, (D, D), 0)
    cc = lax.broadcasted_iota(_I32, (D, D), 1)
    U = (rr <= cc).astype(_F32)
    r1 = lax.broadcasted_iota(_I32, (_PR, _PR), 0)
    r2 = lax.broadcasted_iota(_I32, (_PR, _PR), 1)
    S = (r2 < r1).astype(_F32)
    lane = lax.broadcasted_iota(_I32, (_PR, D), 1)

    def prefix(m):
        P = _dot(m, U)
        off = _dot(S, P[:, D - 1:D])
        return P + off

    qpos = prefix(qm) - 1.0
    qdump = (G + (lane % G)).astype(_F32)
    qt = jnp.where(qm > 0, qpos, qdump).astype(_I32)

    tm = 1.0 - qm
    tpos = prefix(tm) - 1.0
    tdump = ((N - G) + (lane % (_TB - (N - G)))).astype(_F32)
    tt = jnp.where(jnp.logical_and(tm > 0, tpos < (N - G)),
                   tpos, tdump).astype(_I32)
    return qt, tt


_SEL_PER_TILE = 640


def _select_body(logits, qt, tt, qfill, tool_out, query_out,
                 lv, qtv, ttv, tbv, qfv, qb_sh, tb_sh):
    c = lax.axis_index("c")
    s = lax.axis_index("s")

    @pl.when(c == 0)
    def _work():
        base = s * _SEL_PER_TILE
        pltpu.sync_copy(logits.at[pl.ds(base, _SEL_PER_TILE)], lv)
        pltpu.sync_copy(qt.at[pl.ds(base, _SEL_PER_TILE)], qtv)
        pltpu.sync_copy(tt.at[pl.ds(base, _SEL_PER_TILE)], ttv)

        @pl.when(s == 0)
        def _init():
            pltpu.sync_copy(qfill.at[0], qfv)
            pltpu.sync_copy(qfv, qb_sh)

        plsc.subcore_barrier()
        nch = jnp.where(s == NS - 1, 5, _SEL_PER_TILE // CB)

        @pl.loop(0, nch)
        def _scatter(j):
            sl = pl.ds(j * CB, CB)
            pltpu.sync_copy(lv.at[sl], qb_sh.at[qtv.at[sl]])
            pltpu.sync_copy(lv.at[sl], tb_sh.at[ttv.at[sl]])

        plsc.subcore_barrier()

        @pl.when(s == 0)
        def _out():
            pltpu.sync_copy(qb_sh.at[pl.ds(0, G)], qfv.at[pl.ds(0, G)])
            pltpu.sync_copy(qfv.at[pl.ds(0, G)], query_out)
            pltpu.sync_copy(tb_sh.at[pl.ds(0, N - G)], tbv)
            pltpu.sync_copy(tbv, tool_out)


def _sc_select(logits, qt, tt, qfill):
    f = pl.kernel(
        _select_body,
        out_type=(jax.ShapeDtypeStruct((N - G,), _F32),
                  jax.ShapeDtypeStruct((G,), _F32)),
        mesh=_mesh(),
        scratch_types=[
            pltpu.VMEM((_SEL_PER_TILE,), _F32),
            pltpu.VMEM((_SEL_PER_TILE,), _I32),
            pltpu.VMEM((_SEL_PER_TILE,), _I32),
            pltpu.VMEM((N - G,), _F32),
            pltpu.VMEM((D,), _F32),
            pltpu.VMEM_SHARED((2 * G,), _F32),
            pltpu.VMEM_SHARED((_TB,), _F32),
        ],
    )
    return f(logits, qt, tt, qfill)


_BM = 512


def _dot(a, b):
    return lax.dot_general(a, b, (((1,), (0,)), ((), ())),
                           preferred_element_type=_F32)


def _prep_body(b2_r, bp2_r, qt_r, tt_r):
    qt, tt = _prep_positions(b2_r[...], bp2_r[...])
    qt_r[...] = qt
    tt_r[...] = tt


def _tc_prep(b2, bp2):
    return pl.pallas_call(
        _prep_body,
        grid=(1,),
        in_specs=[
            pl.BlockSpec((_PR, D), lambda i: (0, 0)),
            pl.BlockSpec((_PR, D), lambda i: (0, 0)),
        ],
        out_specs=(pl.BlockSpec((_PR, D), lambda i: (0, 0)),
                   pl.BlockSpec((_PR, D), lambda i: (0, 0))),
        out_shape=(jax.ShapeDtypeStruct((_PR, D), _I32),
                   jax.ShapeDtypeStruct((_PR, D), _I32)),
    )(b2, bp2)


def _tc1a_body(x_r, wa_r, ba_r, w0_r, h_r, hw_r):
    x = x_r[...]
    h = x + _dot(x, wa_r[...]) + ba_r[...]
    h_r[...] = h
    hw_r[...] = _dot(h, w0_r[...])


def _tc1a(xp, Wa, ba, W0):
    grid = (NP // _BM,)
    return pl.pallas_call(
        _tc1a_body,
        grid=grid,
        in_specs=[
            pl.BlockSpec((_BM, D), lambda i: (i, 0)),
            pl.BlockSpec((D, D), lambda i: (0, 0)),
            pl.BlockSpec((1, D), lambda i: (0, 0)),
            pl.BlockSpec((D, D), lambda i: (0, 0)),
        ],
        out_specs=(pl.BlockSpec((_BM, D), lambda i: (i, 0)),
                   pl.BlockSpec((_BM, D), lambda i: (i, 0))),
        out_shape=(jax.ShapeDtypeStruct((NP, D), _F32),
                   jax.ShapeDtypeStruct((NP, D), _F32)),
    )(xp, Wa, ba, W0)


def _tc1b_body(hw_r, dv_r, o_r):
    o_r[...] = hw_r[...] * dv_r[...]


def _tc1b(hwraw, dinvc):
    grid = (NP // _BM,)
    return pl.pallas_call(
        _tc1b_body,
        grid=grid,
        in_specs=[
            pl.BlockSpec((_BM, D), lambda i: (i, 0)),
            pl.BlockSpec((_BM, 1), lambda i: (i, 0)),
        ],
        out_specs=pl.BlockSpec((_BM, D), lambda i: (i, 0)),
        out_shape=jax.ShapeDtypeStruct((NP, D), _F32),
    )(hwraw, dinvc)


def _tc2_body(h_r, hwp_r, a0_r, a1_r, dv_r, b_r, w_r, h1_r, hw1_r):
    dv = dv_r[...]
    g = dv * (a0_r[...] + a1_r[...] + hwp_r[...]) + b_r[...]
    g = jnp.maximum(g, 0.0)
    h1 = h_r[...] + g
    h1_r[...] = h1
    hw1_r[...] = _dot(h1, w_r[...]) * dv


def _tc2(h, hwp, a0, a1, dinvc, b, W):
    grid = (NP // _BM,)
    return pl.pallas_call(
        _tc2_body,
        grid=grid,
        in_specs=[
            pl.BlockSpec((_BM, D), lambda i: (i, 0)),
            pl.BlockSpec((_BM, D), lambda i: (i, 0)),
            pl.BlockSpec((_BM, D), lambda i: (i, 0)),
            pl.BlockSpec((_BM, D), lambda i: (i, 0)),
            pl.BlockSpec((_BM, 1), lambda i: (i, 0)),
            pl.BlockSpec((1, D), lambda i: (0, 0)),
            pl.BlockSpec((D, D), lambda i: (0, 0)),
        ],
        out_specs=(pl.BlockSpec((_BM, D), lambda i: (i, 0)),
                   pl.BlockSpec((_BM, D), lambda i: (i, 0))),
        out_shape=(jax.ShapeDtypeStruct((NP, D), _F32),
                   jax.ShapeDtypeStruct((NP, D), _F32)),
    )(h, hwp, a0, a1, dinvc, b, W)


def _tc3_body(h1_r, hw1_r, a0_r, a1_r, dv_r, b_r, wo_r, bo_r, lg_r, qf_r):
    dv = dv_r[...]
    g = dv * (a0_r[...] + a1_r[...] + hw1_r[...]) + b_r[...]
    g = jnp.maximum(g, 0.0)
    h2 = h1_r[...] + g
    lg = _dot(h2, wo_r[...]) + bo_r[...]
    lg_r[...] = lg

    @pl.when(pl.program_id(0) == 0)
    def _qf():
        qf_r[...] = lg[0:1, 0:1] * jnp.ones((1, D), _F32)


def _tc3(h1, hwp1, a0, a1, dinvc, b, Wout, bout):
    grid = (NP // _BM,)
    return pl.pallas_call(
        _tc3_body,
        grid=grid,
        in_specs=[
            pl.BlockSpec((_BM, D), lambda i: (i, 0)),
            pl.BlockSpec((_BM, D), lambda i: (i, 0)),
            pl.BlockSpec((_BM, D), lambda i: (i, 0)),
            pl.BlockSpec((_BM, D), lambda i: (i, 0)),
            pl.BlockSpec((_BM, 1), lambda i: (i, 0)),
            pl.BlockSpec((1, D), lambda i: (0, 0)),
            pl.BlockSpec((D, 1), lambda i: (0, 0)),
            pl.BlockSpec((1, 1), lambda i: (0, 0)),
        ],
        out_specs=(pl.BlockSpec((_BM, 1), lambda i: (i, 0)),
                   pl.BlockSpec((1, D), lambda i: (0, 0))),
        out_shape=(jax.ShapeDtypeStruct((NP, 1), _F32),
                   jax.ShapeDtypeStruct((1, D), _F32)),
    )(h1, hwp1, a0, a1, dinvc, b, Wout, bout)


def kernel(x, W_align, b_align, W_g0, b_g0, W_g1, b_g1, W_out, b_out,
           edge_index, batch_idx, num_graphs):
    xp = jnp.pad(x, ((0, NP - N), (0, 0)))
    src1 = edge_index[0]
    dst1 = edge_index[1]
    bpad = jnp.pad(batch_idx, (0, NP - N), mode="edge")
    bprev = jnp.concatenate(
        [jnp.full((1,), -1, batch_idx.dtype), bpad[:-1]])

    dinv = _sc_deg(dst1)
    qt2, tt2 = _tc_prep(bpad.reshape(_PR, D), bprev.reshape(_PR, D))
    h, hw0raw = _tc1a(xp, W_align, b_align.reshape(1, D), W_g0)

    dinvc = dinv.reshape(NP, 1)
    hwp0 = _tc1b(hw0raw, dinvc)
    a00, a01 = _sc_edge(hwp0, src1, dst1)
    h1, hwp1 = _tc2(h, hwp0, a00, a01, dinvc, b_g0.reshape(1, D), W_g1)
    a10, a11 = _sc_edge(hwp1, src1, dst1)
    logits, qfill = _tc3(h1, hwp1, a10, a11, dinvc, b_g1.reshape(1, D),
                         W_out, b_out.reshape(1, 1))

    tool, query = _sc_select(logits.reshape(NP), qt2.reshape(NP),
                             tt2.reshape(NP), qfill)
    return (tool, query)


# speedup vs baseline: 1.0836x; 1.0703x over previous
import functools

import jax
import jax.numpy as jnp
from jax import lax
from jax.experimental import pallas as pl
from jax.experimental.pallas import tpu as pltpu
from jax.experimental.pallas import tpu_sc as plsc

N = 10000
E = 320000
D = 128
G = 64
NP = 10240
CB = 80
CE = 80
NC = 2
NS = 16
NW = NC * NS
EW = E // NW
ED = E // NS
NCH = EW // CE
ROWS_PER_TILE = NP // NS
ROWS_PER_WORKER = NP // NW

_F32 = jnp.float32
_I32 = jnp.int32


def _mesh():
    return plsc.VectorSubcoreMesh(core_axis_name="c", subcore_axis_name="s")


def _rsqrt_newton(d):
    i = lax.bitcast_convert_type(d, _I32)
    i = jnp.int32(0x5F3759DF) - (i >> 1)
    y = lax.bitcast_convert_type(i, _F32)
    for _ in range(3):
        y = y * (1.5 - 0.5 * d * y * y)
    return y


def _deg_body(dst1, dinv_out, didx_v, ones_v, zv, degbuf_v, dv, deg_sh, sdeg):
    c = lax.axis_index("c")
    s = lax.axis_index("s")
    wid = s * NC + c

    @pl.loop(0, 40)
    def _zero_zv(k):
        zv[pl.ds(k * 16, 16)] = jnp.zeros((16,), _F32)

    @pl.loop(0, 5)
    def _ones(k):
        ones_v[pl.ds(k * 16, 16)] = jnp.ones((16,), _F32)

    pltpu.sync_copy(zv, deg_sh.at[pl.ds(s * ROWS_PER_TILE, ROWS_PER_TILE)])
    pltpu.sync_copy(dst1.at[pl.ds(s * ED, ED)], didx_v)
    plsc.subcore_barrier()

    @pl.loop(0, ED // CB // 10)
    def _scatter(t):
        for p in range(10):
            j = t * 10 + p
            pltpu.async_copy(ones_v, deg_sh.at[didx_v.at[pl.ds(j * CB, CB)]],
                             sdeg, add=True)
        for p in range(10):
            pltpu.make_async_copy(
                ones_v, deg_sh.at[didx_v.at[pl.ds(0, CB)]], sdeg).wait()

    plsc.subcore_barrier()

    base = wid * ROWS_PER_WORKER
    pltpu.sync_copy(deg_sh.at[pl.ds(base, ROWS_PER_WORKER)], degbuf_v)

    @pl.loop(0, ROWS_PER_WORKER // 16)
    def _conv(k):
        d = degbuf_v[pl.ds(k * 16, 16)] + 1.0
        dv[pl.ds(k * 16, 16)] = _rsqrt_newton(d)

    pltpu.sync_copy(dv, dinv_out.at[pl.ds(base, ROWS_PER_WORKER)])


def _sc_deg(dst1):
    f = pl.kernel(
        _deg_body,
        out_type=jax.ShapeDtypeStruct((NP,), _F32),
        mesh=_mesh(),
        scratch_types=[
            pltpu.VMEM((ED,), _I32),
            pltpu.VMEM((CB,), _F32),
            pltpu.VMEM((ROWS_PER_TILE,), _F32),
            pltpu.VMEM((ROWS_PER_WORKER,), _F32),
            pltpu.VMEM((ROWS_PER_WORKER,), _F32),
            pltpu.VMEM_SHARED((NP,), _F32),
            pltpu.SemaphoreType.DMA,
        ],
    )
    return f(dst1)


_SEG = ((0, 64), (64, 61))


def _edge_body(hw, src1, dst1, agg0, agg1, sidx, didx,
               buf0, buf1, buf2, agg_sh,
               sg0, sg1, sg2, ss0, ss1, ss2):
    c = lax.axis_index("c")
    s = lax.axis_index("s")
    wid = s * NC + c
    bufs = (buf0, buf1, buf2)
    sg = (sg0, sg1, sg2)
    ss = (ss0, ss1, ss2)

    @pl.loop(0, CE)
    def _zero_rows(r):
        for k in range(D // 16):
            buf0[r, pl.ds(k * 16, 16)] = jnp.zeros((16,), _F32)

    @pl.loop(0, ROWS_PER_TILE // CE)
    def _zero_agg(k):
        pltpu.async_copy(buf0,
                         agg_sh.at[pl.ds(s * ROWS_PER_TILE + k * CE, CE)],
                         ss0)

    @pl.loop(0, ROWS_PER_TILE // CE)
    def _zero_wait(k):
        pltpu.make_async_copy(
            buf0, agg_sh.at[pl.ds(s * ROWS_PER_TILE, CE)], ss0).wait()

    base = wid * EW

    def start_g(j, p):
        pltpu.async_copy(hw.at[sidx.at[pl.ds(j * CE, CE)]], bufs[p], sg[p])

    def wait_g(p):
        pltpu.make_async_copy(hw.at[sidx.at[pl.ds(0, CE)]], bufs[p],
                              sg[p]).wait()

    def start_s(j, p):
        pltpu.async_copy(bufs[p], agg_sh.at[didx.at[pl.ds(j * CE, CE)]],
                         ss[p], add=True)

    def wait_s(p):
        pltpu.make_async_copy(bufs[p], agg_sh.at[didx.at[pl.ds(0, CE)]],
                              ss[p]).wait()

    for seg, (cb0, nch) in enumerate(_SEG):
        ln = nch * CE
        pltpu.sync_copy(src1.at[pl.ds(base + cb0 * CE, ln)],
                        sidx.at[pl.ds(0, ln)])
        pltpu.sync_copy(dst1.at[pl.ds(base + cb0 * CE, ln)],
                        didx.at[pl.ds(0, ln)])
        start_g(0, 0)
        start_g(1, 1)
        if seg == 0:
            plsc.subcore_barrier()
        wait_g(0); start_s(0, 0); start_g(2, 2)

        @pl.loop(0, (nch - 4) // 3)
        def _main(t):
            j0 = t * 3
            for i, (p, q) in enumerate(((1, 0), (2, 1), (0, 2))):
                j = j0 + 1 + i
                wait_g(p); start_s(j, p)
                wait_s(q); start_g(j + 2, q)

        for j in range(nch - 3, nch):
            p = j % 3
            wait_g(p); start_s(j, p)
            if j + 2 < nch:
                q = (j + 2) % 3
                wait_s(q); start_g(j + 2, q)
        for p in range(3):
            wait_s(p)

    plsc.subcore_barrier()

    src_slice = agg_sh.at[pl.ds(s * ROWS_PER_TILE, ROWS_PER_TILE)]

    @pl.when(c == 0)
    def _w0():
        pltpu.sync_copy(src_slice, agg0.at[pl.ds(s * ROWS_PER_TILE, ROWS_PER_TILE)])

    @pl.when(c == 1)
    def _w1():
        pltpu.sync_copy(src_slice, agg1.at[pl.ds(s * ROWS_PER_TILE, ROWS_PER_TILE)])


def _sc_edge(hw, src1, dst1):
    f = pl.kernel(
        _edge_body,
        out_type=(jax.ShapeDtypeStruct((NP, D), _F32),
                  jax.ShapeDtypeStruct((NP, D), _F32)),
        mesh=_mesh(),
        scratch_types=[
            pltpu.VMEM((_SEG[0][1] * CE,), _I32),
            pltpu.VMEM((_SEG[0][1] * CE,), _I32),
            pltpu.VMEM((CE, D), _F32),
            pltpu.VMEM((CE, D), _F32),
            pltpu.VMEM((CE, D), _F32),
            pltpu.VMEM_SHARED((NP, D), _F32),
        ] + [pltpu.SemaphoreType.DMA] * 6,
    )
    return f(hw, src1, dst1)


_PR = NP // D
_TB = NP


def _prep_positions(b2, bp2):
    qm = (b2 != bp2).astype(_F32)
    rr = lax.broadcasted_iota(_I32, (D, D), 0)
    cc = lax.broadcasted_iota(_I32, (D, D), 1)
    U = (rr <= cc).astype(_F32)
    r1 = lax.broadcasted_iota(_I32, (_PR, _PR), 0)
    r2 = lax.broadcasted_iota(_I32, (_PR, _PR), 1)
    S = (r2 < r1).astype(_F32)
    lane = lax.broadcasted_iota(_I32, (_PR, D), 1)

    def prefix(m):
        P = _dot(m, U)
        off = _dot(S, P[:, D - 1:D])
        return P + off

    qpos = prefix(qm) - 1.0
    qdump = (G + (lane % G)).astype(_F32)
    qt = jnp.where(qm > 0, qpos, qdump).astype(_I32)

    tm = 1.0 - qm
    tpos = prefix(tm) - 1.0
    tdump = ((N - G) + (lane % (_TB - (N - G)))).astype(_F32)
    tt = jnp.where(jnp.logical_and(tm > 0, tpos < (N - G)),
                   tpos, tdump).astype(_I32)
    return qt, tt


_SEL_PER_TILE = 640


def _select_body(logits, qt, tt, qfill, tool_out, query_out,
                 lv, qtv, ttv, tbv, qfv, qb_sh, tb_sh):
    c = lax.axis_index("c")
    s = lax.axis_index("s")

    @pl.when(c == 0)
    def _work():
        base = s * _SEL_PER_TILE
        pltpu.sync_copy(logits.at[pl.ds(base, _SEL_PER_TILE)], lv)
        pltpu.sync_copy(qt.at[pl.ds(base, _SEL_PER_TILE)], qtv)
        pltpu.sync_copy(tt.at[pl.ds(base, _SEL_PER_TILE)], ttv)

        @pl.when(s == 0)
        def _init():
            pltpu.sync_copy(qfill.at[0], qfv)
            pltpu.sync_copy(qfv, qb_sh)

        plsc.subcore_barrier()
        nch = jnp.where(s == NS - 1, 5, _SEL_PER_TILE // CB)

        @pl.loop(0, nch)
        def _scatter(j):
            sl = pl.ds(j * CB, CB)
            pltpu.sync_copy(lv.at[sl], qb_sh.at[qtv.at[sl]])
            pltpu.sync_copy(lv.at[sl], tb_sh.at[ttv.at[sl]])

        plsc.subcore_barrier()

        @pl.when(s == 0)
        def _out():
            pltpu.sync_copy(qb_sh.at[pl.ds(0, G)], qfv.at[pl.ds(0, G)])
            pltpu.sync_copy(qfv.at[pl.ds(0, G)], query_out)
            pltpu.sync_copy(tb_sh.at[pl.ds(0, N - G)], tbv)
            pltpu.sync_copy(tbv, tool_out)


def _sc_select(logits, qt, tt, qfill):
    f = pl.kernel(
        _select_body,
        out_type=(jax.ShapeDtypeStruct((N - G,), _F32),
                  jax.ShapeDtypeStruct((G,), _F32)),
        mesh=_mesh(),
        scratch_types=[
            pltpu.VMEM((_SEL_PER_TILE,), _F32),
            pltpu.VMEM((_SEL_PER_TILE,), _I32),
            pltpu.VMEM((_SEL_PER_TILE,), _I32),
            pltpu.VMEM((N - G,), _F32),
            pltpu.VMEM((D,), _F32),
            pltpu.VMEM_SHARED((2 * G,), _F32),
            pltpu.VMEM_SHARED((_TB,), _F32),
        ],
    )
    return f(logits, qt, tt, qfill)


_BM = 1024


def _dot(a, b):
    return lax.dot_general(a, b, (((1,), (0,)), ((), ())),
                           preferred_element_type=_F32)


def _prep_body(b2_r, bp2_r, qt_r, tt_r):
    qt, tt = _prep_positions(b2_r[...], bp2_r[...])
    qt_r[...] = qt
    tt_r[...] = tt


def _tc_prep(b2, bp2):
    return pl.pallas_call(
        _prep_body,
        grid=(1,),
        in_specs=[
            pl.BlockSpec((_PR, D), lambda i: (0, 0)),
            pl.BlockSpec((_PR, D), lambda i: (0, 0)),
        ],
        out_specs=(pl.BlockSpec((_PR, D), lambda i: (0, 0)),
                   pl.BlockSpec((_PR, D), lambda i: (0, 0))),
        out_shape=(jax.ShapeDtypeStruct((_PR, D), _I32),
                   jax.ShapeDtypeStruct((_PR, D), _I32)),
    )(b2, bp2)


def _tc1a_body(x_r, wa_r, ba_r, w0_r, h_r, hw_r):
    x = x_r[...]
    h = x + _dot(x, wa_r[...]) + ba_r[...]
    h_r[...] = h
    hw_r[...] = _dot(h, w0_r[...])


def _tc1a(xp, Wa, ba, W0):
    grid = (NP // _BM,)
    return pl.pallas_call(
        _tc1a_body,
        grid=grid,
        in_specs=[
            pl.BlockSpec((_BM, D), lambda i: (i, 0)),
            pl.BlockSpec((D, D), lambda i: (0, 0)),
            pl.BlockSpec((1, D), lambda i: (0, 0)),
            pl.BlockSpec((D, D), lambda i: (0, 0)),
        ],
        out_specs=(pl.BlockSpec((_BM, D), lambda i: (i, 0)),
                   pl.BlockSpec((_BM, D), lambda i: (i, 0))),
        out_shape=(jax.ShapeDtypeStruct((NP, D), _F32),
                   jax.ShapeDtypeStruct((NP, D), _F32)),
    )(xp, Wa, ba, W0)


def _tc1b_body(hw_r, dv_r, o_r):
    o_r[...] = hw_r[...] * dv_r[...]


def _tc1b(hwraw, dinvc):
    grid = (NP // _BM,)
    return pl.pallas_call(
        _tc1b_body,
        grid=grid,
        in_specs=[
            pl.BlockSpec((_BM, D), lambda i: (i, 0)),
            pl.BlockSpec((_BM, 1), lambda i: (i, 0)),
        ],
        out_specs=pl.BlockSpec((_BM, D), lambda i: (i, 0)),
        out_shape=jax.ShapeDtypeStruct((NP, D), _F32),
    )(hwraw, dinvc)


def _tc2_body(h_r, hwp_r, a0_r, a1_r, dv_r, b_r, w_r, h1_r, hw1_r):
    dv = dv_r[...]
    g = dv * (a0_r[...] + a1_r[...] + hwp_r[...]) + b_r[...]
    g = jnp.maximum(g, 0.0)
    h1 = h_r[...] + g
    h1_r[...] = h1
    hw1_r[...] = _dot(h1, w_r[...]) * dv


def _tc2(h, hwp, a0, a1, dinvc, b, W):
    grid = (NP // _BM,)
    return pl.pallas_call(
        _tc2_body,
        grid=grid,
        in_specs=[
            pl.BlockSpec((_BM, D), lambda i: (i, 0)),
            pl.BlockSpec((_BM, D), lambda i: (i, 0)),
            pl.BlockSpec((_BM, D), lambda i: (i, 0)),
            pl.BlockSpec((_BM, D), lambda i: (i, 0)),
            pl.BlockSpec((_BM, 1), lambda i: (i, 0)),
            pl.BlockSpec((1, D), lambda i: (0, 0)),
            pl.BlockSpec((D, D), lambda i: (0, 0)),
        ],
        out_specs=(pl.BlockSpec((_BM, D), lambda i: (i, 0)),
                   pl.BlockSpec((_BM, D), lambda i: (i, 0))),
        out_shape=(jax.ShapeDtypeStruct((NP, D), _F32),
                   jax.ShapeDtypeStruct((NP, D), _F32)),
    )(h, hwp, a0, a1, dinvc, b, W)


def _tc3_body(h1_r, hw1_r, a0_r, a1_r, dv_r, b_r, wo_r, bo_r, lg_r, qf_r):
    dv = dv_r[...]
    g = dv * (a0_r[...] + a1_r[...] + hw1_r[...]) + b_r[...]
    g = jnp.maximum(g, 0.0)
    h2 = h1_r[...] + g
    lg = _dot(h2, wo_r[...]) + bo_r[...]
    lg_r[...] = lg

    @pl.when(pl.program_id(0) == 0)
    def _qf():
        qf_r[...] = lg[0:1, 0:1] * jnp.ones((1, D), _F32)


def _tc3(h1, hwp1, a0, a1, dinvc, b, Wout, bout):
    grid = (NP // _BM,)
    return pl.pallas_call(
        _tc3_body,
        grid=grid,
        in_specs=[
            pl.BlockSpec((_BM, D), lambda i: (i, 0)),
            pl.BlockSpec((_BM, D), lambda i: (i, 0)),
            pl.BlockSpec((_BM, D), lambda i: (i, 0)),
            pl.BlockSpec((_BM, D), lambda i: (i, 0)),
            pl.BlockSpec((_BM, 1), lambda i: (i, 0)),
            pl.BlockSpec((1, D), lambda i: (0, 0)),
            pl.BlockSpec((D, 1), lambda i: (0, 0)),
            pl.BlockSpec((1, 1), lambda i: (0, 0)),
        ],
        out_specs=(pl.BlockSpec((_BM, 1), lambda i: (i, 0)),
                   pl.BlockSpec((1, D), lambda i: (0, 0))),
        out_shape=(jax.ShapeDtypeStruct((NP, 1), _F32),
                   jax.ShapeDtypeStruct((1, D), _F32)),
    )(h1, hwp1, a0, a1, dinvc, b, Wout, bout)


def kernel(x, W_align, b_align, W_g0, b_g0, W_g1, b_g1, W_out, b_out,
           edge_index, batch_idx, num_graphs):
    xp = jnp.pad(x, ((0, NP - N), (0, 0)))
    src1 = edge_index[0]
    dst1 = edge_index[1]
    bpad = jnp.pad(batch_idx, (0, NP - N), mode="edge")
    bprev = jnp.concatenate(
        [jnp.full((1,), -1, batch_idx.dtype), bpad[:-1]])

    dinv = _sc_deg(dst1)
    qt2, tt2 = _tc_prep(bpad.reshape(_PR, D), bprev.reshape(_PR, D))
    h, hw0raw = _tc1a(xp, W_align, b_align.reshape(1, D), W_g0)

    dinvc = dinv.reshape(NP, 1)
    hwp0 = _tc1b(hw0raw, dinvc)
    a00, a01 = _sc_edge(hwp0, src1, dst1)
    h1, hwp1 = _tc2(h, hwp0, a00, a01, dinvc, b_g0.reshape(1, D), W_g1)
    a10, a11 = _sc_edge(hwp1, src1, dst1)
    logits, qfill = _tc3(h1, hwp1, a10, a11, dinvc, b_g1.reshape(1, D),
                         W_out, b_out.reshape(1, 1))

    tool, query = _sc_select(logits.reshape(NP), qt2.reshape(NP),
                             tt2.reshape(NP), qfill)
    return (tool, query)


# speedup vs baseline: 1.1123x; 1.0264x over previous
import functools

import jax
import jax.numpy as jnp
from jax import lax
from jax.experimental import pallas as pl
from jax.experimental.pallas import tpu as pltpu
from jax.experimental.pallas import tpu_sc as plsc

N = 10000
E = 320000
D = 128
G = 64
NP = 10240
CB = 80
CE = 80
NC = 2
NS = 16
NW = NC * NS
EW = E // NW
ED = E // NS
NCH = EW // CE
ROWS_PER_TILE = NP // NS
ROWS_PER_WORKER = NP // NW

_F32 = jnp.float32
_I32 = jnp.int32


def _mesh():
    return plsc.VectorSubcoreMesh(core_axis_name="c", subcore_axis_name="s")


def _rsqrt_newton(d):
    i = lax.bitcast_convert_type(d, _I32)
    i = jnp.int32(0x5F3759DF) - (i >> 1)
    y = lax.bitcast_convert_type(i, _F32)
    for _ in range(3):
        y = y * (1.5 - 0.5 * d * y * y)
    return y


def _deg_body(dst1, dinv_out, didx_v, ones_v, zv, degbuf_v, dv, deg_sh, sdeg):
    c = lax.axis_index("c")
    s = lax.axis_index("s")
    wid = s * NC + c

    @pl.loop(0, 40)
    def _zero_zv(k):
        zv[pl.ds(k * 16, 16)] = jnp.zeros((16,), _F32)

    @pl.loop(0, 5)
    def _ones(k):
        ones_v[pl.ds(k * 16, 16)] = jnp.ones((16,), _F32)

    pltpu.sync_copy(zv, deg_sh.at[pl.ds(s * ROWS_PER_TILE, ROWS_PER_TILE)])
    pltpu.sync_copy(dst1.at[pl.ds(s * ED, ED)], didx_v)
    plsc.subcore_barrier()

    @pl.loop(0, ED // CB // 10)
    def _scatter(t):
        for p in range(10):
            j = t * 10 + p
            pltpu.async_copy(ones_v, deg_sh.at[didx_v.at[pl.ds(j * CB, CB)]],
                             sdeg, add=True)
        for p in range(10):
            pltpu.make_async_copy(
                ones_v, deg_sh.at[didx_v.at[pl.ds(0, CB)]], sdeg).wait()

    plsc.subcore_barrier()

    base = wid * ROWS_PER_WORKER
    pltpu.sync_copy(deg_sh.at[pl.ds(base, ROWS_PER_WORKER)], degbuf_v)

    @pl.loop(0, ROWS_PER_WORKER // 16)
    def _conv(k):
        d = degbuf_v[pl.ds(k * 16, 16)] + 1.0
        dv[pl.ds(k * 16, 16)] = _rsqrt_newton(d)

    pltpu.sync_copy(dv, dinv_out.at[pl.ds(base, ROWS_PER_WORKER)])


def _sc_deg(dst1):
    f = pl.kernel(
        _deg_body,
        out_type=jax.ShapeDtypeStruct((NP,), _F32),
        mesh=_mesh(),
        scratch_types=[
            pltpu.VMEM((ED,), _I32),
            pltpu.VMEM((CB,), _F32),
            pltpu.VMEM((ROWS_PER_TILE,), _F32),
            pltpu.VMEM((ROWS_PER_WORKER,), _F32),
            pltpu.VMEM((ROWS_PER_WORKER,), _F32),
            pltpu.VMEM_SHARED((NP,), _F32),
            pltpu.SemaphoreType.DMA,
        ],
    )
    return f(dst1)


_SEG = ((0, 64), (64, 61))


def _edge_body(hw, src1, dst1, agg0, agg1, sidx, didx,
               buf0, buf1, buf2, agg_sh,
               sg0, sg1, sg2, ss0, ss1, ss2):
    c = lax.axis_index("c")
    s = lax.axis_index("s")
    wid = s * NC + c
    bufs = (buf0, buf1, buf2)
    sg = (sg0, sg1, sg2)
    ss = (ss0, ss1, ss2)

    @pl.loop(0, CE)
    def _zero_rows(r):
        for k in range(D // 16):
            buf0[r, pl.ds(k * 16, 16)] = jnp.zeros((16,), _F32)

    @pl.loop(0, ROWS_PER_TILE // CE)
    def _zero_agg(k):
        pltpu.async_copy(buf0,
                         agg_sh.at[pl.ds(s * ROWS_PER_TILE + k * CE, CE)],
                         ss0)

    @pl.loop(0, ROWS_PER_TILE // CE)
    def _zero_wait(k):
        pltpu.make_async_copy(
            buf0, agg_sh.at[pl.ds(s * ROWS_PER_TILE, CE)], ss0).wait()

    base = wid * EW

    def start_g(j, p):
        pltpu.async_copy(hw.at[sidx.at[pl.ds(j * CE, CE)]], bufs[p], sg[p])

    def wait_g(p):
        pltpu.make_async_copy(hw.at[sidx.at[pl.ds(0, CE)]], bufs[p],
                              sg[p]).wait()

    def start_s(j, p):
        pltpu.async_copy(bufs[p], agg_sh.at[didx.at[pl.ds(j * CE, CE)]],
                         ss[p], add=True)

    def wait_s(p):
        pltpu.make_async_copy(bufs[p], agg_sh.at[didx.at[pl.ds(0, CE)]],
                              ss[p]).wait()

    for seg, (cb0, nch) in enumerate(_SEG):
        ln = nch * CE
        pltpu.sync_copy(src1.at[pl.ds(base + cb0 * CE, ln)],
                        sidx.at[pl.ds(0, ln)])
        pltpu.sync_copy(dst1.at[pl.ds(base + cb0 * CE, ln)],
                        didx.at[pl.ds(0, ln)])
        start_g(0, 0)
        start_g(1, 1)
        if seg == 0:
            plsc.subcore_barrier()
        wait_g(0); start_s(0, 0); start_g(2, 2)

        @pl.loop(0, (nch - 4) // 3)
        def _main(t):
            j0 = t * 3
            for i, (p, q) in enumerate(((1, 0), (2, 1), (0, 2))):
                j = j0 + 1 + i
                wait_g(p); start_s(j, p)
                wait_s(q); start_g(j + 2, q)

        for j in range(nch - 3, nch):
            p = j % 3
            wait_g(p); start_s(j, p)
            if j + 2 < nch:
                q = (j + 2) % 3
                wait_s(q); start_g(j + 2, q)
        for p in range(3):
            wait_s(p)

    plsc.subcore_barrier()

    src_slice = agg_sh.at[pl.ds(s * ROWS_PER_TILE, ROWS_PER_TILE)]

    @pl.when(c == 0)
    def _w0():
        pltpu.sync_copy(src_slice, agg0.at[pl.ds(s * ROWS_PER_TILE, ROWS_PER_TILE)])

    @pl.when(c == 1)
    def _w1():
        pltpu.sync_copy(src_slice, agg1.at[pl.ds(s * ROWS_PER_TILE, ROWS_PER_TILE)])


def _sc_edge(hw, src1, dst1):
    f = pl.kernel(
        _edge_body,
        out_type=(jax.ShapeDtypeStruct((NP, D), _F32),
                  jax.ShapeDtypeStruct((NP, D), _F32)),
        mesh=_mesh(),
        scratch_types=[
            pltpu.VMEM((_SEG[0][1] * CE,), _I32),
            pltpu.VMEM((_SEG[0][1] * CE,), _I32),
            pltpu.VMEM((CE, D), _F32),
            pltpu.VMEM((CE, D), _F32),
            pltpu.VMEM((CE, D), _F32),
            pltpu.VMEM_SHARED((NP, D), _F32),
        ] + [pltpu.SemaphoreType.DMA] * 6,
    )
    return f(hw, src1, dst1)


_PR = NP // D
_TB = NP


def _prep_positions(b2, bp2):
    qm = (b2 != bp2).astype(_F32)
    rr = lax.broadcasted_iota(_I32, (D, D), 0)
    cc = lax.broadcasted_iota(_I32, (D, D), 1)
    U = (rr <= cc).astype(_F32)
    r1 = lax.broadcasted_iota(_I32, (_PR, _PR), 0)
    r2 = lax.broadcasted_iota(_I32, (_PR, _PR), 1)
    S = (r2 < r1).astype(_F32)
    lane = lax.broadcasted_iota(_I32, (_PR, D), 1)

    def prefix(m):
        P = _dot(m, U)
        off = _dot(S, P[:, D - 1:D])
        return P + off

    qpos = prefix(qm) - 1.0
    qdump = (G + (lane % G)).astype(_F32)
    qt = jnp.where(qm > 0, qpos, qdump).astype(_I32)

    tm = 1.0 - qm
    tpos = prefix(tm) - 1.0
    tdump = ((N - G) + (lane % (_TB - (N - G)))).astype(_F32)
    tt = jnp.where(jnp.logical_and(tm > 0, tpos < (N - G)),
                   tpos, tdump).astype(_I32)
    return qt, tt


_SEL_PER_TILE = 640


def _select_body(logits, qt, tt, qfill, tool_out, query_out,
                 lv, qtv, ttv, tbv, qfv, qb_sh, tb_sh):
    c = lax.axis_index("c")
    s = lax.axis_index("s")

    @pl.when(c == 0)
    def _work():
        base = s * _SEL_PER_TILE
        pltpu.sync_copy(logits.at[pl.ds(base, _SEL_PER_TILE)], lv)
        pltpu.sync_copy(qt.at[pl.ds(base, _SEL_PER_TILE)], qtv)
        pltpu.sync_copy(tt.at[pl.ds(base, _SEL_PER_TILE)], ttv)

        @pl.when(s == 0)
        def _init():
            pltpu.sync_copy(qfill.at[0], qfv)
            pltpu.sync_copy(qfv, qb_sh)

        plsc.subcore_barrier()
        nch = jnp.where(s == NS - 1, 5, _SEL_PER_TILE // CB)

        @pl.loop(0, nch)
        def _scatter(j):
            sl = pl.ds(j * CB, CB)
            pltpu.sync_copy(lv.at[sl], qb_sh.at[qtv.at[sl]])
            pltpu.sync_copy(lv.at[sl], tb_sh.at[ttv.at[sl]])

        plsc.subcore_barrier()

        @pl.when(s == 0)
        def _out():
            pltpu.sync_copy(qb_sh.at[pl.ds(0, G)], qfv.at[pl.ds(0, G)])
            pltpu.sync_copy(qfv.at[pl.ds(0, G)], query_out)
            pltpu.sync_copy(tb_sh.at[pl.ds(0, N - G)], tbv)
            pltpu.sync_copy(tbv, tool_out)


def _sc_select(logits, qt, tt, qfill):
    f = pl.kernel(
        _select_body,
        out_type=(jax.ShapeDtypeStruct((N - G,), _F32),
                  jax.ShapeDtypeStruct((G,), _F32)),
        mesh=_mesh(),
        scratch_types=[
            pltpu.VMEM((_SEL_PER_TILE,), _F32),
            pltpu.VMEM((_SEL_PER_TILE,), _I32),
            pltpu.VMEM((_SEL_PER_TILE,), _I32),
            pltpu.VMEM((N - G,), _F32),
            pltpu.VMEM((D,), _F32),
            pltpu.VMEM_SHARED((2 * G,), _F32),
            pltpu.VMEM_SHARED((_TB,), _F32),
        ],
    )
    return f(logits, qt, tt, qfill)


_BM = 2048


def _dot(a, b):
    return lax.dot_general(a, b, (((1,), (0,)), ((), ())),
                           preferred_element_type=_F32)


def _prep_body(b2_r, bp2_r, qt_r, tt_r):
    qt, tt = _prep_positions(b2_r[...], bp2_r[...])
    qt_r[...] = qt
    tt_r[...] = tt


def _tc_prep(b2, bp2):
    return pl.pallas_call(
        _prep_body,
        grid=(1,),
        in_specs=[
            pl.BlockSpec((_PR, D), lambda i: (0, 0)),
            pl.BlockSpec((_PR, D), lambda i: (0, 0)),
        ],
        out_specs=(pl.BlockSpec((_PR, D), lambda i: (0, 0)),
                   pl.BlockSpec((_PR, D), lambda i: (0, 0))),
        out_shape=(jax.ShapeDtypeStruct((_PR, D), _I32),
                   jax.ShapeDtypeStruct((_PR, D), _I32)),
    )(b2, bp2)


def _tc1a_body(x_r, wa_r, ba_r, w0_r, h_r, hw_r):
    x = x_r[...]
    h = x + _dot(x, wa_r[...]) + ba_r[...]
    h_r[...] = h
    hw_r[...] = _dot(h, w0_r[...])


def _tc1a(xp, Wa, ba, W0):
    grid = (NP // _BM,)
    return pl.pallas_call(
        _tc1a_body,
        grid=grid,
        in_specs=[
            pl.BlockSpec((_BM, D), lambda i: (i, 0)),
            pl.BlockSpec((D, D), lambda i: (0, 0)),
            pl.BlockSpec((1, D), lambda i: (0, 0)),
            pl.BlockSpec((D, D), lambda i: (0, 0)),
        ],
        out_specs=(pl.BlockSpec((_BM, D), lambda i: (i, 0)),
                   pl.BlockSpec((_BM, D), lambda i: (i, 0))),
        out_shape=(jax.ShapeDtypeStruct((NP, D), _F32),
                   jax.ShapeDtypeStruct((NP, D), _F32)),
    )(xp, Wa, ba, W0)


def _tc1b_body(hw_r, dv_r, o_r):
    o_r[...] = hw_r[...] * dv_r[...]


def _tc1b(hwraw, dinvc):
    grid = (NP // _BM,)
    return pl.pallas_call(
        _tc1b_body,
        grid=grid,
        in_specs=[
            pl.BlockSpec((_BM, D), lambda i: (i, 0)),
            pl.BlockSpec((_BM, 1), lambda i: (i, 0)),
        ],
        out_specs=pl.BlockSpec((_BM, D), lambda i: (i, 0)),
        out_shape=jax.ShapeDtypeStruct((NP, D), _F32),
    )(hwraw, dinvc)


def _tc2_body(h_r, hwp_r, a0_r, a1_r, dv_r, b_r, w_r, h1_r, hw1_r):
    dv = dv_r[...]
    g = dv * (a0_r[...] + a1_r[...] + hwp_r[...]) + b_r[...]
    g = jnp.maximum(g, 0.0)
    h1 = h_r[...] + g
    h1_r[...] = h1
    hw1_r[...] = _dot(h1, w_r[...]) * dv


def _tc2(h, hwp, a0, a1, dinvc, b, W):
    grid = (NP // _BM,)
    return pl.pallas_call(
        _tc2_body,
        grid=grid,
        in_specs=[
            pl.BlockSpec((_BM, D), lambda i: (i, 0)),
            pl.BlockSpec((_BM, D), lambda i: (i, 0)),
            pl.BlockSpec((_BM, D), lambda i: (i, 0)),
            pl.BlockSpec((_BM, D), lambda i: (i, 0)),
            pl.BlockSpec((_BM, 1), lambda i: (i, 0)),
            pl.BlockSpec((1, D), lambda i: (0, 0)),
            pl.BlockSpec((D, D), lambda i: (0, 0)),
        ],
        out_specs=(pl.BlockSpec((_BM, D), lambda i: (i, 0)),
                   pl.BlockSpec((_BM, D), lambda i: (i, 0))),
        out_shape=(jax.ShapeDtypeStruct((NP, D), _F32),
                   jax.ShapeDtypeStruct((NP, D), _F32)),
    )(h, hwp, a0, a1, dinvc, b, W)


def _tc3_body(h1_r, hw1_r, a0_r, a1_r, dv_r, b_r, wo_r, bo_r, lg_r, qf_r):
    dv = dv_r[...]
    g = dv * (a0_r[...] + a1_r[...] + hw1_r[...]) + b_r[...]
    g = jnp.maximum(g, 0.0)
    h2 = h1_r[...] + g
    lg = _dot(h2, wo_r[...]) + bo_r[...]
    lg_r[...] = lg

    @pl.when(pl.program_id(0) == 0)
    def _qf():
        qf_r[...] = lg[0:1, 0:1] * jnp.ones((1, D), _F32)


def _tc3(h1, hwp1, a0, a1, dinvc, b, Wout, bout):
    grid = (NP // _BM,)
    return pl.pallas_call(
        _tc3_body,
        grid=grid,
        in_specs=[
            pl.BlockSpec((_BM, D), lambda i: (i, 0)),
            pl.BlockSpec((_BM, D), lambda i: (i, 0)),
            pl.BlockSpec((_BM, D), lambda i: (i, 0)),
            pl.BlockSpec((_BM, D), lambda i: (i, 0)),
            pl.BlockSpec((_BM, 1), lambda i: (i, 0)),
            pl.BlockSpec((1, D), lambda i: (0, 0)),
            pl.BlockSpec((D, 1), lambda i: (0, 0)),
            pl.BlockSpec((1, 1), lambda i: (0, 0)),
        ],
        out_specs=(pl.BlockSpec((_BM, 1), lambda i: (i, 0)),
                   pl.BlockSpec((1, D), lambda i: (0, 0))),
        out_shape=(jax.ShapeDtypeStruct((NP, 1), _F32),
                   jax.ShapeDtypeStruct((1, D), _F32)),
    )(h1, hwp1, a0, a1, dinvc, b, Wout, bout)


def kernel(x, W_align, b_align, W_g0, b_g0, W_g1, b_g1, W_out, b_out,
           edge_index, batch_idx, num_graphs):
    xp = jnp.pad(x, ((0, NP - N), (0, 0)))
    src1 = edge_index[0]
    dst1 = edge_index[1]
    bpad = jnp.pad(batch_idx, (0, NP - N), mode="edge")
    bprev = jnp.concatenate(
        [jnp.full((1,), -1, batch_idx.dtype), bpad[:-1]])

    dinv = _sc_deg(dst1)
    qt2, tt2 = _tc_prep(bpad.reshape(_PR, D), bprev.reshape(_PR, D))
    h, hw0raw = _tc1a(xp, W_align, b_align.reshape(1, D), W_g0)

    dinvc = dinv.reshape(NP, 1)
    hwp0 = _tc1b(hw0raw, dinvc)
    a00, a01 = _sc_edge(hwp0, src1, dst1)
    h1, hwp1 = _tc2(h, hwp0, a00, a01, dinvc, b_g0.reshape(1, D), W_g1)
    a10, a11 = _sc_edge(hwp1, src1, dst1)
    logits, qfill = _tc3(h1, hwp1, a10, a11, dinvc, b_g1.reshape(1, D),
                         W_out, b_out.reshape(1, 1))

    tool, query = _sc_select(logits.reshape(NP), qt2.reshape(NP),
                             tt2.reshape(NP), qfill)
    return (tool, query)
